# baseline (speedup 1.0000x reference)
import functools

import jax
import jax.numpy as jnp
from jax import lax
from jax.experimental import pallas as pl
from jax.experimental.pallas import tpu as pltpu
from jax.experimental.pallas import tpu_sc as plsc

BATCH = 16384
EMBED_DIM = 64
N_ROWS = 1_000_000

_info = plsc.get_sparse_core_info()
_NC, _NS, _L = _info.num_cores, _info.num_subcores, _info.num_lanes
_NW = _NC * _NS
_B_PER_W = BATCH // _NW
_STREAM = 128

_mesh = plsc.VectorSubcoreMesh(core_axis_name="c", subcore_axis_name="s")


@functools.partial(
    pl.kernel,
    mesh=_mesh,
    compiler_params=pltpu.CompilerParams(use_tc_tiling_on_sc=False),
    out_type=jax.ShapeDtypeStruct((BATCH, EMBED_DIM), jnp.float32),
    scratch_types=[
        pltpu.VMEM((_B_PER_W,), jnp.int32),
        pltpu.VMEM((_B_PER_W, EMBED_DIM), jnp.float32),
        pltpu.SemaphoreType.DMA,
        pltpu.SemaphoreType.DMA,
    ],
)
def _lookup_one(idx_hbm, tab, out_hbm, idx_v, rows_v, sem_g, sem_o):
    wid = lax.axis_index("s") * _NC + lax.axis_index("c")
    base = wid * _B_PER_W
    pltpu.sync_copy(idx_hbm.at[pl.ds(base, _B_PER_W)], idx_v)
    copies = [
        pltpu.async_copy(
            tab.at[idx_v.at[pl.ds(k * _STREAM, _STREAM)]],
            rows_v.at[pl.ds(k * _STREAM, _STREAM)], sem_g)
        for k in range(_B_PER_W // _STREAM)
    ]
    for c in copies:
        c.wait()
    pltpu.async_copy(
        rows_v, out_hbm.at[pl.ds(base, _B_PER_W)], sem_o).wait()


def kernel(user, item, user_table, item_table):
    u = _lookup_one(user, user_table)
    i = _lookup_one(item, item_table)
    return (u, i)


# speedup vs baseline: 1.1144x; 1.1144x over previous
import functools

import jax
import jax.numpy as jnp
from jax import lax
from jax.experimental import pallas as pl
from jax.experimental.pallas import tpu as pltpu
from jax.experimental.pallas import tpu_sc as plsc

BATCH = 16384
EMBED_DIM = 64
N_ROWS = 1_000_000

_info = plsc.get_sparse_core_info()
_NC, _NS, _L = _info.num_cores, _info.num_subcores, _info.num_lanes
_NW = _NC * _NS
_B_PER_W = BATCH // _NW
_CHUNK = 32
_STREAM = 128

_mesh = plsc.VectorSubcoreMesh(core_axis_name="c", subcore_axis_name="s")


@functools.partial(
    pl.kernel,
    mesh=_mesh,
    compiler_params=pltpu.CompilerParams(
        use_tc_tiling_on_sc=True, needs_layout_passes=False),
    out_type=jax.ShapeDtypeStruct((8, 8, BATCH), jnp.float32),
    scratch_types=[
        pltpu.VMEM((_B_PER_W,), jnp.int32),
        pltpu.VMEM((_CHUNK, 8, EMBED_DIM), jnp.float32),
        pltpu.VMEM((8, 8, _B_PER_W), jnp.float32),
        pltpu.SemaphoreType.DMA,
        pltpu.SemaphoreType.DMA,
    ],
)
def _lookup_tiled(idx_hbm, tab, out8, idx_v, buf, stage, sem_g, sem_o):
    wid = lax.axis_index("s") * _NC + lax.axis_index("c")
    base = wid * _B_PER_W
    pltpu.sync_copy(idx_hbm.at[pl.ds(base, _B_PER_W)], idx_v)

    def step(g, carry):
        copies = []
        svecs = []
        for sub in range(_CHUNK // _L):
            rvec = idx_v[pl.ds(g * _CHUNK + sub * _L, _L)]
            r0vec = rvec & jnp.int32(~7)
            svecs.append(rvec & jnp.int32(7))
            for k in range(_L):
                j = sub * _L + k
                r0k = pl.multiple_of(r0vec[k], 8)
                copies.append(pltpu.async_copy(
                    tab.at[pl.ds(r0k, 8), :], buf.at[j], sem_g))
        for c in copies:
            c.wait()
        for sub in range(_CHUNK // _L):
            jvec = lax.iota(jnp.int32, _L) + sub * _L
            svec = svecs[sub]
            off = g * _CHUNK + sub * _L
            for a in range(8):
                for b2 in range(8):
                    cvec = jnp.full((_L,), 8 * a + b2, jnp.int32)
                    vals = plsc.load_gather(buf, [jvec, svec, cvec])
                    stage[a, b2, pl.ds(off, _L)] = vals
        return carry
    lax.fori_loop(0, _B_PER_W // _CHUNK, step, 0)
    pltpu.async_copy(
        stage, out8.at[:, :, pl.ds(base, _B_PER_W)], sem_o).wait()


@functools.partial(
    pl.kernel,
    mesh=_mesh,
    compiler_params=pltpu.CompilerParams(use_tc_tiling_on_sc=False),
    out_type=jax.ShapeDtypeStruct((BATCH, EMBED_DIM), jnp.float32),
    scratch_types=[
        pltpu.VMEM((_B_PER_W,), jnp.int32),
        pltpu.VMEM((_B_PER_W, EMBED_DIM), jnp.float32),
        pltpu.SemaphoreType.DMA,
        pltpu.SemaphoreType.DMA,
    ],
)
def _lookup_linear(idx_hbm, tab, out_hbm, idx_v, rows_v, sem_g, sem_o):
    wid = lax.axis_index("s") * _NC + lax.axis_index("c")
    base = wid * _B_PER_W
    pltpu.sync_copy(idx_hbm.at[pl.ds(base, _B_PER_W)], idx_v)
    copies = [
        pltpu.async_copy(
            tab.at[idx_v.at[pl.ds(k * _STREAM, _STREAM)]],
            rows_v.at[pl.ds(k * _STREAM, _STREAM)], sem_g)
        for k in range(_B_PER_W // _STREAM)
    ]
    for c in copies:
        c.wait()
    pltpu.async_copy(
        rows_v, out_hbm.at[pl.ds(base, _B_PER_W)], sem_o).wait()


def kernel(user, item, user_table, item_table):
    io = _lookup_linear(item, item_table)
    uo8 = _lookup_tiled(user, user_table)
    return (uo8.reshape(EMBED_DIM, BATCH).T, io)


# speedup vs baseline: 1.4811x; 1.3291x over previous
import functools

import jax
import jax.numpy as jnp
from jax import lax
from jax.experimental import pallas as pl
from jax.experimental.pallas import tpu as pltpu
from jax.experimental.pallas import tpu_sc as plsc

BATCH = 16384
EMBED_DIM = 64
N_ROWS = 1_000_000

_info = plsc.get_sparse_core_info()
_NC, _NS, _L = _info.num_cores, _info.num_subcores, _info.num_lanes
_NW = _NC * _NS
_B_PER_W = BATCH // _NW
_CHUNK = 32

_mesh = plsc.VectorSubcoreMesh(core_axis_name="c", subcore_axis_name="s")


@functools.partial(
    pl.kernel,
    mesh=_mesh,
    compiler_params=pltpu.CompilerParams(
        use_tc_tiling_on_sc=True, needs_layout_passes=False),
    out_type=jax.ShapeDtypeStruct((8, 8, BATCH), jnp.float32),
    scratch_types=[
        pltpu.VMEM((_B_PER_W,), jnp.int32),
        pltpu.VMEM((_CHUNK, 8, EMBED_DIM), jnp.float32),
        pltpu.VMEM((8, 8, _B_PER_W), jnp.float32),
        pltpu.SemaphoreType.DMA,
        pltpu.SemaphoreType.DMA,
    ],
)
def _lookup_one(idx_hbm, tab, out8, idx_v, buf, stage, sem_g, sem_o):
    wid = lax.axis_index("s") * _NC + lax.axis_index("c")
    base = wid * _B_PER_W
    pltpu.sync_copy(idx_hbm.at[pl.ds(base, _B_PER_W)], idx_v)

    def step(g, carry):
        copies = []
        svecs = []
        for sub in range(_CHUNK // _L):
            rvec = idx_v[pl.ds(g * _CHUNK + sub * _L, _L)]
            r0vec = rvec & jnp.int32(~7)
            svecs.append(rvec & jnp.int32(7))
            for k in range(_L):
                j = sub * _L + k
                r0k = pl.multiple_of(r0vec[k], 8)
                copies.append(pltpu.async_copy(
                    tab.at[pl.ds(r0k, 8), :], buf.at[j], sem_g))
        for c in copies:
            c.wait()
        for sub in range(_CHUNK // _L):
            jvec = lax.iota(jnp.int32, _L) + sub * _L
            svec = svecs[sub]
            off = g * _CHUNK + sub * _L
            for a in range(8):
                for b2 in range(8):
                    cvec = jnp.full((_L,), 8 * a + b2, jnp.int32)
                    vals = plsc.load_gather(buf, [jvec, svec, cvec])
                    stage[a, b2, pl.ds(off, _L)] = vals
        return carry
    lax.fori_loop(0, _B_PER_W // _CHUNK, step, 0)
    pltpu.async_copy(
        stage, out8.at[:, :, pl.ds(base, _B_PER_W)], sem_o).wait()


def kernel(user, item, user_table, item_table):
    uo8 = _lookup_one(user, user_table)
    io8 = _lookup_one(item, item_table)
    return (uo8.reshape(EMBED_DIM, BATCH).T, io8.reshape(EMBED_DIM, BATCH).T)


# speedup vs baseline: 1.4846x; 1.0024x over previous
import functools

import jax
import jax.numpy as jnp
from jax import lax
from jax.experimental import pallas as pl
from jax.experimental.pallas import tpu as pltpu
from jax.experimental.pallas import tpu_sc as plsc

BATCH = 16384
EMBED_DIM = 64
N_ROWS = 1_000_000

_info = plsc.get_sparse_core_info()
_NC, _NS, _L = _info.num_cores, _info.num_subcores, _info.num_lanes
_NW = _NC * _NS
_B_PER_W = BATCH // _NW
_CHUNK = 64

_mesh = plsc.VectorSubcoreMesh(core_axis_name="c", subcore_axis_name="s")


@functools.partial(
    pl.kernel,
    mesh=_mesh,
    compiler_params=pltpu.CompilerParams(
        use_tc_tiling_on_sc=True, needs_layout_passes=False),
    out_type=jax.ShapeDtypeStruct((8, 8, BATCH), jnp.float32),
    scratch_types=[
        pltpu.VMEM((_B_PER_W,), jnp.int32),
        pltpu.VMEM((_CHUNK, 8, EMBED_DIM), jnp.float32),
        pltpu.VMEM((8, 8, _B_PER_W), jnp.float32),
        pltpu.SemaphoreType.DMA,
        pltpu.SemaphoreType.DMA,
    ],
)
def _lookup_one(idx_hbm, tab, out8, idx_v, buf, stage, sem_g, sem_o):
    wid = lax.axis_index("s") * _NC + lax.axis_index("c")
    base = wid * _B_PER_W
    pltpu.sync_copy(idx_hbm.at[pl.ds(base, _B_PER_W)], idx_v)

    def step(g, carry):
        copies = []
        svecs = []
        for sub in range(_CHUNK // _L):
            rvec = idx_v[pl.ds(g * _CHUNK + sub * _L, _L)]
            r0vec = rvec & jnp.int32(~7)
            svecs.append(rvec & jnp.int32(7))
            for k in range(_L):
                j = sub * _L + k
                r0k = pl.multiple_of(r0vec[k], 8)
                copies.append(pltpu.async_copy(
                    tab.at[pl.ds(r0k, 8), :], buf.at[j], sem_g))
        for c in copies:
            c.wait()
        for sub in range(_CHUNK // _L):
            jvec = lax.iota(jnp.int32, _L) + sub * _L
            svec = svecs[sub]
            off = g * _CHUNK + sub * _L
            for a in range(8):
                for b2 in range(8):
                    cvec = jnp.full((_L,), 8 * a + b2, jnp.int32)
                    vals = plsc.load_gather(buf, [jvec, svec, cvec])
                    stage[a, b2, pl.ds(off, _L)] = vals
        return carry
    lax.fori_loop(0, _B_PER_W // _CHUNK, step, 0)
    pltpu.async_copy(
        stage, out8.at[:, :, pl.ds(base, _B_PER_W)], sem_o).wait()


def kernel(user, item, user_table, item_table):
    uo8 = _lookup_one(user, user_table)
    io8 = _lookup_one(item, item_table)
    return (uo8.reshape(EMBED_DIM, BATCH).T, io8.reshape(EMBED_DIM, BATCH).T)
